# baseline (speedup 1.0000x reference)
import functools

import jax
import jax.numpy as jnp
from jax import lax
from jax.experimental import pallas as pl
from jax.experimental.pallas import tpu as pltpu
from jax.experimental.pallas import tpu_sc as plsc

_NUM_EMB = 1_000_000
_D = 64
_B = 4096 * 200

_NC = 2
_NS = 16
_NW = _NC * _NS

_IDX_PER_GATHER = 128
_K = 8
_CHUNK = _K * _IDX_PER_GATHER
_PER_W = _B // _NW
_NCHUNK = _PER_W // _CHUNK


def _emb_body(tok_hbm, table_hbm, out_hbm, idx_v, rows_v, sem):
    wid = lax.axis_index("s") * _NC + lax.axis_index("c")

    @pl.loop(0, _NCHUNK)
    def _chunk(i):
        pltpu.sync_copy(tok_hbm.at[wid, i], idx_v)
        cps = [
            pltpu.async_copy(table_hbm.at[idx_v.at[j]], rows_v.at[j], sem)
            for j in range(_K)
        ]
        for cp in cps:
            cp.wait()
        pltpu.sync_copy(rows_v, out_hbm.at[wid, i])


_emb = functools.partial(
    pl.kernel,
    out_type=jax.ShapeDtypeStruct((_NW, _NCHUNK, _K, _IDX_PER_GATHER, _D),
                                  jnp.float32),
    mesh=plsc.VectorSubcoreMesh(core_axis_name="c", subcore_axis_name="s"),
    scratch_types=[
        pltpu.VMEM((_K, _IDX_PER_GATHER), jnp.int32),
        pltpu.VMEM((_K, _IDX_PER_GATHER, _D), jnp.float32),
        pltpu.SemaphoreType.DMA,
    ],
    compiler_params=pltpu.CompilerParams(use_tc_tiling_on_sc=False),
)(_emb_body)


def kernel(tokens, embedding_weight):
    batch, seq = tokens.shape
    tok = tokens.astype(jnp.int32).reshape(_NW, _NCHUNK, _K, _IDX_PER_GATHER)
    out = _emb(tok, embedding_weight)
    return out.reshape(batch, seq, _D)


# speedup vs baseline: 1.0173x; 1.0173x over previous
import functools

import jax
import jax.numpy as jnp
from jax import lax
from jax.experimental import pallas as pl
from jax.experimental.pallas import tpu as pltpu
from jax.experimental.pallas import tpu_sc as plsc

_NUM_EMB = 1_000_000
_D = 64
_B = 4096 * 200

_NC = 2
_NS = 16
_NW = _NC * _NS

_IDX_PER_GATHER = 128
_K = 4
_CHUNK = _K * _IDX_PER_GATHER
_PER_W = _B // _NW
_NCHUNK = _PER_W // _CHUNK


def _emb_body(tok_hbm, table_hbm, out_hbm, idx_all, rows_v,
              gsem0, gsem1, wsem0, wsem1):
    wid = lax.axis_index("s") * _NC + lax.axis_index("c")
    gsem = (gsem0, gsem1)
    wsem = (wsem0, wsem1)

    pltpu.sync_copy(tok_hbm.at[wid], idx_all)

    def fire_gathers(chunk, b):
        for j in range(_K):
            pltpu.async_copy(table_hbm.at[idx_all.at[chunk * _K + j]],
                             rows_v.at[b, j], gsem[b])

    def wait_gathers(b):
        pltpu.make_async_copy(out_hbm.at[wid, 0], rows_v.at[b],
                              gsem[b]).wait()

    def fire_wb(chunk, b):
        pltpu.async_copy(rows_v.at[b], out_hbm.at[wid, chunk], wsem[b])

    def wait_wb(b):
        pltpu.make_async_copy(rows_v.at[b], out_hbm.at[wid, 0],
                              wsem[b]).wait()

    fire_gathers(0, 0)

    @pl.loop(0, _NCHUNK, step=2)
    def _chunk(i):
        for b in range(2):
            ic = i + b
            nb = 1 - b

            @pl.when(ic + 1 < _NCHUNK)
            def _():
                @pl.when(ic >= 1)
                def _():
                    wait_wb(nb)
                fire_gathers(ic + 1, nb)

            wait_gathers(b)
            fire_wb(ic, b)

    wait_wb(0)
    wait_wb(1)


_emb = functools.partial(
    pl.kernel,
    out_type=jax.ShapeDtypeStruct((_NW, _NCHUNK, _K, _IDX_PER_GATHER, _D),
                                  jnp.float32),
    mesh=plsc.VectorSubcoreMesh(core_axis_name="c", subcore_axis_name="s"),
    scratch_types=[
        pltpu.VMEM((_NCHUNK * _K, _IDX_PER_GATHER), jnp.int32),
        pltpu.VMEM((2, _K, _IDX_PER_GATHER, _D), jnp.float32),
        pltpu.SemaphoreType.DMA,
        pltpu.SemaphoreType.DMA,
        pltpu.SemaphoreType.DMA,
        pltpu.SemaphoreType.DMA,
    ],
    compiler_params=pltpu.CompilerParams(use_tc_tiling_on_sc=False),
)(_emb_body)


def kernel(tokens, embedding_weight):
    batch, seq = tokens.shape
    tok = tokens.astype(jnp.int32).reshape(_NW, _NCHUNK * _K, _IDX_PER_GATHER)
    out = _emb(tok, embedding_weight)
    return out.reshape(batch, seq, _D)


# speedup vs baseline: 1.3553x; 1.3322x over previous
import functools

import jax
import jax.numpy as jnp
from jax import lax
from jax.experimental import pallas as pl
from jax.experimental.pallas import tpu as pltpu
from jax.experimental.pallas import tpu_sc as plsc

_NUM_EMB = 1_000_000
_D = 64
_DPAD = 128
_B = 4096 * 200

_NC = 2
_NS = 16
_NW = _NC * _NS

_IDX_PER_GATHER = 128
_K = 4
_CHUNK = _K * _IDX_PER_GATHER
_PER_W = _B // _NW
_NCHUNK = _PER_W // _CHUNK


def _emb_body(tok_hbm, table_hbm, out_hbm, idx_all, rows_v,
              gsem0, gsem1, wsem0, wsem1):
    wid = lax.axis_index("s") * _NC + lax.axis_index("c")
    gsem = (gsem0, gsem1)
    wsem = (wsem0, wsem1)
    base = wid * _PER_W

    pltpu.sync_copy(tok_hbm.at[wid], idx_all)

    def fire_gathers(chunk, b):
        for j in range(_K):
            pltpu.async_copy(
                table_hbm.at[idx_all.at[chunk * _K + j]],
                rows_v.at[b, pl.ds(j * _IDX_PER_GATHER, _IDX_PER_GATHER)],
                gsem[b])

    def wait_gathers(b):
        pltpu.make_async_copy(out_hbm.at[pl.ds(0, _CHUNK), pl.ds(0, _D)],
                              rows_v.at[b], gsem[b]).wait()

    def out_slice(chunk):
        return out_hbm.at[pl.ds(base + chunk * _CHUNK, _CHUNK), pl.ds(0, _D)]

    def fire_wb(chunk, b):
        pltpu.async_copy(rows_v.at[b], out_slice(chunk), wsem[b])

    def wait_wb(b):
        pltpu.make_async_copy(rows_v.at[b], out_slice(0), wsem[b]).wait()

    fire_gathers(0, 0)

    @pl.loop(0, _NCHUNK, step=2)
    def _chunk(i):
        for b in range(2):
            ic = i + b
            nb = 1 - b

            @pl.when(ic + 1 < _NCHUNK)
            def _():
                @pl.when(ic >= 1)
                def _():
                    wait_wb(nb)
                fire_gathers(ic + 1, nb)

            wait_gathers(b)
            fire_wb(ic, b)

    wait_wb(0)
    wait_wb(1)


_emb = functools.partial(
    pl.kernel,
    out_type=jax.ShapeDtypeStruct((_B, _DPAD), jnp.float32),
    mesh=plsc.VectorSubcoreMesh(core_axis_name="c", subcore_axis_name="s"),
    scratch_types=[
        pltpu.VMEM((_NCHUNK * _K, _IDX_PER_GATHER), jnp.int32),
        pltpu.VMEM((2, _CHUNK, _D), jnp.float32),
        pltpu.SemaphoreType.DMA,
        pltpu.SemaphoreType.DMA,
        pltpu.SemaphoreType.DMA,
        pltpu.SemaphoreType.DMA,
    ],
    compiler_params=pltpu.CompilerParams(use_tc_tiling_on_sc=False),
)(_emb_body)


def kernel(tokens, embedding_weight):
    batch, seq = tokens.shape
    tok = tokens.astype(jnp.int32).reshape(_NW, _NCHUNK * _K, _IDX_PER_GATHER)
    out = _emb(tok, embedding_weight)
    return out[:, :_D].reshape(batch, seq, _D)
